# Initial kernel scaffold; baseline (speedup 1.0000x reference)
import jax
import jax.numpy as jnp
from jax.experimental import pallas as pl


def kernel(z, pos, batch, emb, Wq, Wk, Wv, Wo, Wf, Wu, Wreg, breg):
    raise NotImplementedError("write your pallas kernel here")


# speedup vs baseline: 46.8900x; 46.8900x over previous
import functools
import math

import jax
import jax.numpy as jnp
from jax.experimental import pallas as pl
from jax.experimental.pallas import tpu as pltpu

CUTOFF = 5.0
D = 32
L = 2
H = 2
DH = D // H
NRBF = 4
ZMAX = 100
NMOL = 100
GAMMA = 10.0 / CUTOFF
CENTERS = tuple(CUTOFF * r / (NRBF - 1) for r in range(NRBF))
INV_SQRT_DH = 1.0 / math.sqrt(DH)

T = 256
CB = 512
F32 = jnp.float32


def _dot(a, b):
    return jnp.dot(a, b, preferred_element_type=F32)


def _embed_proj_body(z_ref, emb_ref, wq_ref, wk_ref, wv_ref,
                     h_ref, q_ref, k_ref, v_ref):
    zt = z_ref[...]
    lanes = jax.lax.broadcasted_iota(jnp.int32, (T, ZMAX), 1)
    oh = (zt == lanes).astype(F32)
    h = _dot(oh, emb_ref[...])
    h_ref[...] = h
    q_ref[...] = _dot(h, wq_ref[...])
    k_ref[...] = _dot(h, wk_ref[...])
    v_ref[...] = _dot(h, wv_ref[...])


def _sweep_body(cb0_ref, ncb_ref, q_ref, posr_ref, br_ref,
                kt3_ref, v3_ref, post3_ref, bt3_ref, wf_ref,
                agg_ref, xagg_ref):
    t = pl.program_id(0)
    row0 = t * T
    qb = q_ref[...]
    pxr = posr_ref[:, 0:1]
    pyr = posr_ref[:, 1:2]
    pzr = posr_ref[:, 2:3]
    br = br_ref[...]
    wf = wf_ref[...]
    agg_ref[...] = jnp.zeros_like(agg_ref)
    xagg_ref[...] = jnp.zeros_like(xagg_ref)
    q0 = qb[:, 0:DH]
    q1 = qb[:, DH:D]
    riota = jax.lax.broadcasted_iota(jnp.int32, (T, CB), 0) + row0
    ciota = jax.lax.broadcasted_iota(jnp.int32, (T, CB), 1)

    def body(jc, carry):
        ktb = kt3_ref[jc]
        vb = v3_ref[jc]
        pc = post3_ref[jc]
        bc = bt3_ref[jc]
        dx = pxr - pc[0:1, :]
        dy = pyr - pc[1:2, :]
        dz = pzr - pc[2:3, :]
        d2 = dx * dx + dy * dy + dz * dz
        dij = jnp.sqrt(d2 + 1e-12)
        inv = 1.0 / dij
        msk = (d2 < CUTOFF * CUTOFF) & (br == bc) & (riota != ciota + jc * CB)
        fc = 0.5 * (jnp.cos(dij * (math.pi / CUTOFF)) + 1.0)
        fcm = jnp.where(msk, fc, 0.0)
        s0 = _dot(q0, ktb[0:DH, :])
        s1 = _dot(q1, ktb[DH:D, :])
        a0 = jax.nn.sigmoid(s0 * INV_SQRT_DH) * fcm
        a1 = jax.nn.sigmoid(s1 * INV_SQRT_DH) * fcm
        ux = dx * inv
        uy = dy * inv
        uz = dz * inv
        m0 = jnp.zeros((T, D), F32)
        m1 = jnp.zeros((T, D), F32)
        xs = [[jnp.zeros((T, D), F32) for _ in range(2)] for _ in range(3)]
        for r in range(NRBF):
            rb = jnp.exp(-GAMMA * (dij - CENTERS[r]) ** 2)
            vfr = vb * wf[r:r + 1, :]
            a0r = a0 * rb
            a1r = a1 * rb
            m0 = m0 + _dot(a0r, vfr)
            m1 = m1 + _dot(a1r, vfr)
            for k, u in enumerate((ux, uy, uz)):
                xs[k][0] = xs[k][0] + _dot(u * a0r, vfr)
                xs[k][1] = xs[k][1] + _dot(u * a1r, vfr)
        agg_ref[...] += jnp.concatenate([m0[:, 0:DH], m1[:, DH:D]], axis=1)
        for k in range(3):
            xagg_ref[k] += jnp.concatenate(
                [xs[k][0][:, 0:DH], xs[k][1][:, DH:D]], axis=1)
        return carry

    lo = cb0_ref[t]
    jax.lax.fori_loop(lo, lo + ncb_ref[t], body, 0)


def _update_proj_body(first, h_ref, agg_ref, xagg_ref, x_ref,
                      wo_ref, wu_ref, wq_ref, wk_ref, wv_ref,
                      ho_ref, xo_ref, q_ref, k_ref, v_ref):
    h = h_ref[...] + _dot(agg_ref[...], wo_ref[...])
    if first:
        x = xagg_ref[...]
    else:
        x = x_ref[...] + xagg_ref[...]
    xo_ref[...] = x
    sq = x[0] * x[0] + x[1] * x[1] + x[2] * x[2]
    h = h + _dot(jnp.tanh(sq), wu_ref[...])
    ho_ref[...] = h
    q_ref[...] = _dot(h, wq_ref[...])
    k_ref[...] = _dot(h, wk_ref[...])
    v_ref[...] = _dot(h, wv_ref[...])


def _final_body(nt, h_ref, agg_ref, xagg_ref, x_ref, bt_ref,
                wo_ref, wu_ref, wreg_ref, breg_ref,
                out_ref, sums_ref, counts_ref):
    t = pl.program_id(0)

    @pl.when(t == 0)
    def _init():
        sums_ref[...] = jnp.zeros_like(sums_ref)
        counts_ref[...] = jnp.zeros_like(counts_ref)

    h = h_ref[...] + _dot(agg_ref[...], wo_ref[...])
    x = x_ref[...] + xagg_ref[...]
    sq = x[0] * x[0] + x[1] * x[1] + x[2] * x[2]
    h = h + _dot(jnp.tanh(sq), wu_ref[...])
    bt = bt_ref[...]
    rows = jax.lax.broadcasted_iota(jnp.int32, (128, T), 0)
    oh = (rows == bt).astype(F32)
    sums_ref[...] += _dot(oh, h)
    counts_ref[...] += jnp.sum(oh, axis=1, keepdims=True)

    @pl.when(t == nt - 1)
    def _fin():
        xm = sums_ref[...] / jnp.maximum(counts_ref[...], 1.0)
        out_ref[...] = _dot(xm, wreg_ref[...]) + breg_ref[...]


def kernel(z, pos, batch, emb, Wq, Wk, Wv, Wo, Wf, Wu, Wreg, breg):
    n = z.shape[0]
    np_ = pl.cdiv(n, CB) * CB
    nt = np_ // T
    nc = np_ // CB
    padn = np_ - n

    z_p = jnp.pad(z.astype(jnp.int32), (0, padn)).reshape(np_, 1)
    batch_p = jnp.pad(batch.astype(jnp.int32), (0, padn),
                      constant_values=NMOL)
    pos_p = jnp.pad(pos, ((0, padn), (0, 0)))
    br = batch_p.reshape(np_, 1)
    bt3 = batch_p.reshape(nc, 1, CB)
    bt_flat = batch_p.reshape(1, np_)
    post3 = pos_p.T.reshape(3, nc, CB).transpose(1, 0, 2)

    starts = batch_p[::T]
    ends = batch_p[T - 1::T]
    lo = jnp.searchsorted(batch_p, starts, side='left').astype(jnp.int32)
    hi = jnp.searchsorted(batch_p, ends, side='right').astype(jnp.int32)
    cb0 = lo // CB
    ncb = ((hi + CB - 1) // CB - cb0).astype(jnp.int32)

    row_spec = pl.BlockSpec((T, D), lambda t: (t, 0))
    row1_spec = pl.BlockSpec((T, 1), lambda t: (t, 0))
    full = lambda shape: pl.BlockSpec(shape, lambda t: (0,) * len(shape))
    x_spec = pl.BlockSpec((3, T, D), lambda t: (0, t, 0))

    h0, q, k, v = pl.pallas_call(
        _embed_proj_body,
        grid=(nt,),
        in_specs=[row1_spec, full((ZMAX, D)), full((D, D)), full((D, D)),
                  full((D, D))],
        out_specs=[row_spec] * 4,
        out_shape=[jax.ShapeDtypeStruct((np_, D), F32)] * 4,
    )(z_p, emb, Wq[0], Wk[0], Wv[0])

    h = h0
    x = None
    for l in range(L):
        kt3 = k.T.reshape(D, nc, CB).transpose(1, 0, 2)
        v3 = v.reshape(nc, CB, D)
        grid_spec = pltpu.PrefetchScalarGridSpec(
            num_scalar_prefetch=2,
            grid=(nt,),
            in_specs=[
                pl.BlockSpec((T, D), lambda t, c0, nb: (t, 0)),
                pl.BlockSpec((T, 3), lambda t, c0, nb: (t, 0)),
                pl.BlockSpec((T, 1), lambda t, c0, nb: (t, 0)),
                pl.BlockSpec((nc, D, CB), lambda t, c0, nb: (0, 0, 0)),
                pl.BlockSpec((nc, CB, D), lambda t, c0, nb: (0, 0, 0)),
                pl.BlockSpec((nc, 3, CB), lambda t, c0, nb: (0, 0, 0)),
                pl.BlockSpec((nc, 1, CB), lambda t, c0, nb: (0, 0, 0)),
                pl.BlockSpec((NRBF, D), lambda t, c0, nb: (0, 0)),
            ],
            out_specs=[
                pl.BlockSpec((T, D), lambda t, c0, nb: (t, 0)),
                pl.BlockSpec((3, T, D), lambda t, c0, nb: (0, t, 0)),
            ],
        )
        agg, xagg = pl.pallas_call(
            _sweep_body,
            grid_spec=grid_spec,
            out_shape=[jax.ShapeDtypeStruct((np_, D), F32),
                       jax.ShapeDtypeStruct((3, np_, D), F32)],
        )(cb0, ncb, q, pos_p, br, kt3, v3, post3, bt3, Wf[l])

        if l < L - 1:
            first = x is None
            ins = [h, agg, xagg, xagg if first else x,
                   Wo[l], Wu[l], Wq[l + 1], Wk[l + 1], Wv[l + 1]]
            h, x, q, k, v = pl.pallas_call(
                functools.partial(_update_proj_body, first),
                grid=(nt,),
                in_specs=[row_spec, row_spec, x_spec, x_spec,
                          full((D, D)), full((D, D)), full((D, D)),
                          full((D, D)), full((D, D))],
                out_specs=[row_spec, x_spec, row_spec, row_spec, row_spec],
                out_shape=[jax.ShapeDtypeStruct((np_, D), F32),
                           jax.ShapeDtypeStruct((3, np_, D), F32)] +
                          [jax.ShapeDtypeStruct((np_, D), F32)] * 3,
            )(*ins)
        else:
            out = pl.pallas_call(
                functools.partial(_final_body, nt),
                grid=(nt,),
                in_specs=[row_spec, row_spec, x_spec, x_spec,
                          pl.BlockSpec((1, T), lambda t: (0, t)),
                          full((D, D)), full((D, D)), full((D, 1)),
                          full((1, 1))],
                out_specs=pl.BlockSpec((128, 1), lambda t: (0, 0)),
                out_shape=jax.ShapeDtypeStruct((128, 1), F32),
                scratch_shapes=[pltpu.VMEM((128, D), F32),
                                pltpu.VMEM((128, 1), F32)],
            )(h, agg, xagg, x, bt_flat, Wo[l], Wu[l], Wreg,
              breg.reshape(1, 1))

    return out[:NMOL, 0]
